# baseline (speedup 1.0000x reference)
import functools

import jax
import jax.numpy as jnp
from jax import lax
from jax.experimental import pallas as pl
from jax.experimental.pallas import tpu as pltpu
from jax.experimental.pallas import tpu_sc as plsc

_BATCH = 16384
_HIST = 200
_EMBED = 32
_B = _BATCH * _HIST

_NC = 2
_NS = 16
_NW = _NC * _NS

_GROUPS = _B // 128
_G_PER_W = _GROUPS // _NW
_GCHUNK = 4
_CHUNK = _GCHUNK * 128
_N_CHUNKS = _G_PER_W // _GCHUNK
_N_PAIRS = _N_CHUNKS // 2


def _embed_kernel(idx_hbm, table_hbm, out_hbm, idx_v, rows_v, embt_v,
                  si0, si1, sg0, sg1, ss0, ss1):
    wid = lax.axis_index("s") * _NC + lax.axis_index("c")
    g_base = wid * _G_PER_W
    si = (si0, si1)
    sg = (sg0, sg1)
    ss = (ss0, ss1)
    iota = lax.broadcasted_iota(jnp.int32, (16,), 0)

    def idx_cp(k, b):
        return pltpu.make_async_copy(
            idx_hbm.at[pl.ds((g_base + k * _GCHUNK) * 128, _CHUNK)],
            idx_v.at[b], si[b])

    def gat_cp(b):
        return pltpu.make_async_copy(
            table_hbm.at[idx_v.at[b]], rows_v.at[b], sg[b])

    def sct_cp(k, b, r):
        g0 = g_base + k * _GCHUNK
        h = g0 // 128
        c0 = g0 % 128
        return pltpu.make_async_copy(
            embt_v.at[b, r], out_hbm.at[h, r, pl.ds(c0, _GCHUNK)], ss[b])

    def transpose(b):
        def r_body(r, carry_r):
            def s_body(s, carry_s):
                col = jnp.full((16,), 8 * r + s, jnp.int32)
                for j in range(_GCHUNK):
                    for k in range(8):
                        row = iota + (j * 128 + 16 * k)
                        vec = plsc.load_gather(rows_v.at[b], [row, col])
                        embt_v[b, r, j, s, pl.ds(16 * k, 16)] = vec
                return carry_s
            return lax.fori_loop(0, 8, s_body, carry_r)
        lax.fori_loop(0, 4, r_body, 0)

    def start_sct(k, b):
        for r in range(4):
            sct_cp(k, b, r).start()

    def wait_sct(b):
        for r in range(4):
            sct_cp(0, b, r).wait()

    idx_cp(0, 0).start()
    idx_cp(1, 1).start()
    idx_cp(0, 0).wait()
    gat_cp(0).start()
    idx_cp(1, 1).wait()
    gat_cp(1).start()
    gat_cp(0).wait()
    transpose(0)
    start_sct(0, 0)
    idx_cp(2, 0).start()
    gat_cp(1).wait()
    transpose(1)
    start_sct(1, 1)
    idx_cp(3, 1).start()

    def body(i, carry):
        k0 = 2 * i
        idx_cp(k0, 0).wait()
        wait_sct(0)
        gat_cp(0).start()
        idx_cp(k0 + 1, 1).wait()
        wait_sct(1)
        gat_cp(1).start()
        gat_cp(0).wait()
        transpose(0)
        start_sct(k0, 0)
        idx_cp(k0 + 2, 0).start()
        gat_cp(1).wait()
        transpose(1)
        start_sct(k0 + 1, 1)
        idx_cp(k0 + 3, 1).start()
        return carry

    lax.fori_loop(1, _N_PAIRS - 1, body, 0)

    k0 = 2 * (_N_PAIRS - 1)
    idx_cp(k0, 0).wait()
    wait_sct(0)
    gat_cp(0).start()
    idx_cp(k0 + 1, 1).wait()
    wait_sct(1)
    gat_cp(1).start()
    gat_cp(0).wait()
    transpose(0)
    start_sct(k0, 0)
    gat_cp(1).wait()
    transpose(1)
    start_sct(k0 + 1, 1)
    wait_sct(0)
    wait_sct(1)


@jax.jit
def _embed(x_flat, table):
    mesh = plsc.VectorSubcoreMesh(core_axis_name="c", subcore_axis_name="s")
    k = functools.partial(
        pl.kernel,
        mesh=mesh,
        out_type=jax.ShapeDtypeStruct((_HIST, 4, 128, 8, 128), jnp.float32),
        scratch_types=[
            pltpu.VMEM((2, _CHUNK), jnp.int32),
            pltpu.VMEM((2, _CHUNK, _EMBED), jnp.float32),
            pltpu.VMEM((2, 4, _GCHUNK, 8, 128), jnp.float32),
            pltpu.SemaphoreType.DMA,
            pltpu.SemaphoreType.DMA,
            pltpu.SemaphoreType.DMA,
            pltpu.SemaphoreType.DMA,
            pltpu.SemaphoreType.DMA,
            pltpu.SemaphoreType.DMA,
        ],
        compiler_params=pltpu.CompilerParams(
            use_tc_tiling_on_sc=False, needs_layout_passes=False),
    )(_embed_kernel)
    return k(x_flat, table)


def kernel(x, table):
    x_flat = x.T.reshape(-1).astype(jnp.int32)
    out = _embed(x_flat, table)
    return out.transpose(2, 4, 0, 1, 3).reshape(_BATCH, _HIST, _EMBED)


# speedup vs baseline: 1.9764x; 1.9764x over previous
import functools

import jax
import jax.numpy as jnp
from jax import lax
from jax.experimental import pallas as pl
from jax.experimental.pallas import tpu as pltpu
from jax.experimental.pallas import tpu_sc as plsc

_BATCH = 16384
_HIST = 200
_EMBED = 32
_B = _BATCH * _HIST

_NC = 2
_NS = 16
_NW = _NC * _NS

_GROUPS = _B // 128
_G_PER_W = _GROUPS // _NW
_GCHUNK = 4
_CHUNK = _GCHUNK * 128
_N_CHUNKS = _G_PER_W // _GCHUNK
_N_PAIRS = _N_CHUNKS // 2
_LPAD = 133


def _embed_kernel(idx_hbm, table_hbm, out_hbm, idx_v, rows_v, embt_v,
                  si0, si1, sg0, sg1, ss0, ss1):
    wid = lax.axis_index("s") * _NC + lax.axis_index("c")
    g_base = wid * _G_PER_W
    si = (si0, si1)
    sg = (sg0, sg1)
    ss = (ss0, ss1)
    iota = lax.broadcasted_iota(jnp.int32, (16,), 0)
    rvec = [(iota + 16 * m) // 8 for m in range(2)]
    svec = [(iota + 16 * m) % 8 for m in range(2)]

    def idx_cp(k, b):
        return pltpu.make_async_copy(
            idx_hbm.at[pl.ds((g_base + k * _GCHUNK) * 128, _CHUNK)],
            idx_v.at[b], si[b])

    def gat_cp(b):
        return pltpu.make_async_copy(
            table_hbm.at[idx_v.at[b]], rows_v.at[b], sg[b])

    def sct_cp(k, b, j, r):
        g0 = g_base + k * _GCHUNK
        h = g0 // 128
        c0 = g0 % 128
        return pltpu.make_async_copy(
            embt_v.at[b, j, r, :, pl.ds(0, 128)],
            out_hbm.at[h, r, c0 + j], ss[b])

    def transpose(b):
        def row_body(i, carry):
            j = i // 128
            l = i % 128
            jv = jnp.full((16,), j, jnp.int32)
            lv = jnp.full((16,), l, jnp.int32)
            for m in range(2):
                vec = rows_v[b, i, pl.ds(16 * m, 16)]
                plsc.store_scatter(embt_v.at[b], [jv, rvec[m], svec[m], lv],
                                   vec)
            return carry
        lax.fori_loop(0, _CHUNK, row_body, 0)

    def start_sct(k, b):
        for j in range(_GCHUNK):
            for r in range(4):
                sct_cp(k, b, j, r).start()

    def wait_sct(b):
        for j in range(_GCHUNK):
            for r in range(4):
                sct_cp(0, b, j, r).wait()

    idx_cp(0, 0).start()
    idx_cp(1, 1).start()
    idx_cp(0, 0).wait()
    gat_cp(0).start()
    idx_cp(1, 1).wait()
    gat_cp(1).start()
    gat_cp(0).wait()
    transpose(0)
    start_sct(0, 0)
    idx_cp(2, 0).start()
    gat_cp(1).wait()
    transpose(1)
    start_sct(1, 1)
    idx_cp(3, 1).start()

    def body(i, carry):
        k0 = 2 * i
        idx_cp(k0, 0).wait()
        wait_sct(0)
        gat_cp(0).start()
        idx_cp(k0 + 1, 1).wait()
        wait_sct(1)
        gat_cp(1).start()
        gat_cp(0).wait()
        transpose(0)
        start_sct(k0, 0)
        idx_cp(k0 + 2, 0).start()
        gat_cp(1).wait()
        transpose(1)
        start_sct(k0 + 1, 1)
        idx_cp(k0 + 3, 1).start()
        return carry

    lax.fori_loop(1, _N_PAIRS - 1, body, 0)

    k0 = 2 * (_N_PAIRS - 1)
    idx_cp(k0, 0).wait()
    wait_sct(0)
    gat_cp(0).start()
    idx_cp(k0 + 1, 1).wait()
    wait_sct(1)
    gat_cp(1).start()
    gat_cp(0).wait()
    transpose(0)
    start_sct(k0, 0)
    gat_cp(1).wait()
    transpose(1)
    start_sct(k0 + 1, 1)
    wait_sct(0)
    wait_sct(1)


@jax.jit
def _embed(x_flat, table):
    mesh = plsc.VectorSubcoreMesh(core_axis_name="c", subcore_axis_name="s")
    k = functools.partial(
        pl.kernel,
        mesh=mesh,
        out_type=jax.ShapeDtypeStruct((_HIST, 4, 128, 8, 128), jnp.float32),
        scratch_types=[
            pltpu.VMEM((2, _CHUNK), jnp.int32),
            pltpu.VMEM((2, _CHUNK, _EMBED), jnp.float32),
            pltpu.VMEM((2, _GCHUNK, 4, 8, _LPAD), jnp.float32),
            pltpu.SemaphoreType.DMA,
            pltpu.SemaphoreType.DMA,
            pltpu.SemaphoreType.DMA,
            pltpu.SemaphoreType.DMA,
            pltpu.SemaphoreType.DMA,
            pltpu.SemaphoreType.DMA,
        ],
        compiler_params=pltpu.CompilerParams(
            use_tc_tiling_on_sc=False, needs_layout_passes=False),
    )(_embed_kernel)
    return k(x_flat, table)


def kernel(x, table):
    x_flat = x.T.reshape(-1).astype(jnp.int32)
    out = _embed(x_flat, table)
    return out.transpose(2, 4, 0, 1, 3).reshape(_BATCH, _HIST, _EMBED)


# speedup vs baseline: 2.0994x; 1.0623x over previous
import functools

import jax
import jax.numpy as jnp
from jax import lax
from jax.experimental import pallas as pl
from jax.experimental.pallas import tpu as pltpu
from jax.experimental.pallas import tpu_sc as plsc

_BATCH = 16384
_HIST = 200
_EMBED = 32
_B = _BATCH * _HIST

_NC = 2
_NS = 16
_NW = _NC * _NS

_GROUPS = _B // 128
_G_PER_W = _GROUPS // _NW
_GCHUNK = 4
_CHUNK = _GCHUNK * 128
_N_CHUNKS = _G_PER_W // _GCHUNK
_N_PAIRS = _N_CHUNKS // 2
_LPAD = 133


def _embed_kernel(idx_hbm, table_hbm, out_hbm, idx_v, rows_v, embt_v,
                  si0, si1, sg0, sg1, ss0, ss1):
    wid = lax.axis_index("s") * _NC + lax.axis_index("c")
    g_base = wid * _G_PER_W
    si = (si0, si1)
    sg = (sg0, sg1)
    ss = (ss0, ss1)
    iota = lax.broadcasted_iota(jnp.int32, (16,), 0)
    rvec = [(iota + 16 * m) // 8 for m in range(2)]
    svec = [(iota + 16 * m) % 8 for m in range(2)]

    def idx_cp(k, b):
        return pltpu.make_async_copy(
            idx_hbm.at[pl.ds((g_base + k * _GCHUNK) * 128, _CHUNK)],
            idx_v.at[b], si[b])

    def gat_cp(b):
        return pltpu.make_async_copy(
            table_hbm.at[idx_v.at[b]], rows_v.at[b], sg[b])

    def sct_cp(k, b, j):
        g0 = g_base + k * _GCHUNK
        h = g0 // 128
        c0 = g0 % 128
        return pltpu.make_async_copy(
            embt_v.at[b, j, :, :, pl.ds(0, 128)],
            out_hbm.at[h, :, c0 + j], ss[b])

    def transpose(b):
        for j in range(_GCHUNK):
            def l_body(q, carry, j=j):
                for dl in range(4):
                    l = q * 4 + dl
                    lv = jnp.full((16,), l, jnp.int32)
                    for m in range(2):
                        vec = rows_v[b, j * 128 + l, pl.ds(16 * m, 16)]
                        plsc.store_scatter(
                            embt_v.at[b, j], [rvec[m], svec[m], lv], vec)
                return carry
            lax.fori_loop(0, 32, l_body, 0)

    def start_sct(k, b):
        for j in range(_GCHUNK):
            sct_cp(k, b, j).start()

    def wait_sct(b):
        for j in range(_GCHUNK):
            sct_cp(0, b, j).wait()

    idx_cp(0, 0).start()
    idx_cp(1, 1).start()
    idx_cp(0, 0).wait()
    gat_cp(0).start()
    idx_cp(1, 1).wait()
    gat_cp(1).start()
    gat_cp(0).wait()
    transpose(0)
    start_sct(0, 0)
    idx_cp(2, 0).start()
    gat_cp(1).wait()
    transpose(1)
    start_sct(1, 1)
    idx_cp(3, 1).start()

    def body(i, carry):
        k0 = 2 * i
        idx_cp(k0, 0).wait()
        wait_sct(0)
        gat_cp(0).start()
        idx_cp(k0 + 1, 1).wait()
        wait_sct(1)
        gat_cp(1).start()
        gat_cp(0).wait()
        transpose(0)
        start_sct(k0, 0)
        idx_cp(k0 + 2, 0).start()
        gat_cp(1).wait()
        transpose(1)
        start_sct(k0 + 1, 1)
        idx_cp(k0 + 3, 1).start()
        return carry

    lax.fori_loop(1, _N_PAIRS - 1, body, 0)

    k0 = 2 * (_N_PAIRS - 1)
    idx_cp(k0, 0).wait()
    wait_sct(0)
    gat_cp(0).start()
    idx_cp(k0 + 1, 1).wait()
    wait_sct(1)
    gat_cp(1).start()
    gat_cp(0).wait()
    transpose(0)
    start_sct(k0, 0)
    gat_cp(1).wait()
    transpose(1)
    start_sct(k0 + 1, 1)
    wait_sct(0)
    wait_sct(1)


@jax.jit
def _embed(x_flat, table):
    mesh = plsc.VectorSubcoreMesh(core_axis_name="c", subcore_axis_name="s")
    k = functools.partial(
        pl.kernel,
        mesh=mesh,
        out_type=jax.ShapeDtypeStruct((_HIST, 4, 128, 8, 128), jnp.float32),
        scratch_types=[
            pltpu.VMEM((2, _CHUNK), jnp.int32),
            pltpu.VMEM((2, _CHUNK, _EMBED), jnp.float32),
            pltpu.VMEM((2, _GCHUNK, 4, 8, _LPAD), jnp.float32),
            pltpu.SemaphoreType.DMA,
            pltpu.SemaphoreType.DMA,
            pltpu.SemaphoreType.DMA,
            pltpu.SemaphoreType.DMA,
            pltpu.SemaphoreType.DMA,
            pltpu.SemaphoreType.DMA,
        ],
        compiler_params=pltpu.CompilerParams(
            use_tc_tiling_on_sc=False, needs_layout_passes=False),
    )(_embed_kernel)
    return k(x_flat, table)


def kernel(x, table):
    x_flat = x.T.reshape(-1).astype(jnp.int32)
    out = _embed(x_flat, table)
    return out.transpose(2, 4, 0, 1, 3).reshape(_BATCH, _HIST, _EMBED)


# speedup vs baseline: 3.1197x; 1.4860x over previous
import functools

import jax
import jax.numpy as jnp
from jax import lax
from jax.experimental import pallas as pl
from jax.experimental.pallas import tpu as pltpu
from jax.experimental.pallas import tpu_sc as plsc

_BATCH = 16384
_HIST = 200
_EMBED = 32
_B = _BATCH * _HIST

_NC = 2
_NS = 16
_NW = _NC * _NS

_GROUPS = _B // 128
_G_PER_W = _GROUPS // _NW
_GCHUNK = 4
_CHUNK = _GCHUNK * 128
_N_CHUNKS = _G_PER_W // _GCHUNK
_N_PAIRS = _N_CHUNKS // 2
_LPAD = 133


def _embed_kernel(idx_hbm, table_hbm, out_hbm, idx_v, rows_v, embt_v,
                  si0, si1, sg0, sg1, ss0, ss1):
    wid = lax.axis_index("s") * _NC + lax.axis_index("c")
    g_base = wid * _G_PER_W
    si = (si0, si1)
    sg = (sg0, sg1)
    ss = (ss0, ss1)
    iota = lax.broadcasted_iota(jnp.int32, (16,), 0)
    rvec = [(iota + 16 * m) // 8 for m in range(2)]
    svec = [(iota + 16 * m) % 8 for m in range(2)]

    def idx_cp(k, b):
        return pltpu.make_async_copy(
            idx_hbm.at[pl.ds((g_base + k * _GCHUNK) * 128, _CHUNK)],
            idx_v.at[b], si[b])

    def gat_cp(b):
        return pltpu.make_async_copy(
            table_hbm.at[idx_v.at[b]], rows_v.at[b], sg[b])

    def sct_cp(k, b, j):
        g0 = g_base + k * _GCHUNK
        h = g0 // 128
        c0 = g0 % 128
        return pltpu.make_async_copy(
            embt_v.at[b, j, :, :, pl.ds(0, 128)],
            out_hbm.at[h, :, c0 + j], ss[b])

    sidx = [rvec[m] * 1064 + svec[m] * 133 for m in range(2)]
    zero = jnp.zeros((16,), jnp.int32)

    def transpose(b):
        for j in range(_GCHUNK):
            init = tuple(sidx[m] + dl for dl in range(4) for m in range(2))
            def l_body(q, idxs, j=j):
                vecs = []
                for dl in range(4):
                    l = q * 4 + dl
                    for m in range(2):
                        vecs.append(rows_v[b, j * 128 + l, pl.ds(16 * m, 16)])
                for t in range(8):
                    plsc.store_scatter(embt_v.at[b, j],
                                       [zero, zero, idxs[t]], vecs[t])
                return tuple(v + 4 for v in idxs)
            lax.fori_loop(0, 32, l_body, init)

    def start_sct(k, b):
        for j in range(_GCHUNK):
            sct_cp(k, b, j).start()

    def wait_sct(b):
        for j in range(_GCHUNK):
            sct_cp(0, b, j).wait()

    idx_cp(0, 0).start()
    idx_cp(1, 1).start()
    idx_cp(0, 0).wait()
    gat_cp(0).start()
    idx_cp(1, 1).wait()
    gat_cp(1).start()
    gat_cp(0).wait()
    transpose(0)
    start_sct(0, 0)
    idx_cp(2, 0).start()
    gat_cp(1).wait()
    transpose(1)
    start_sct(1, 1)
    idx_cp(3, 1).start()

    def body(i, carry):
        k0 = 2 * i
        idx_cp(k0, 0).wait()
        wait_sct(0)
        gat_cp(0).start()
        idx_cp(k0 + 1, 1).wait()
        wait_sct(1)
        gat_cp(1).start()
        gat_cp(0).wait()
        transpose(0)
        start_sct(k0, 0)
        idx_cp(k0 + 2, 0).start()
        gat_cp(1).wait()
        transpose(1)
        start_sct(k0 + 1, 1)
        idx_cp(k0 + 3, 1).start()
        return carry

    lax.fori_loop(1, _N_PAIRS - 1, body, 0)

    k0 = 2 * (_N_PAIRS - 1)
    idx_cp(k0, 0).wait()
    wait_sct(0)
    gat_cp(0).start()
    idx_cp(k0 + 1, 1).wait()
    wait_sct(1)
    gat_cp(1).start()
    gat_cp(0).wait()
    transpose(0)
    start_sct(k0, 0)
    gat_cp(1).wait()
    transpose(1)
    start_sct(k0 + 1, 1)
    wait_sct(0)
    wait_sct(1)


@jax.jit
def _embed(x_flat, table):
    mesh = plsc.VectorSubcoreMesh(core_axis_name="c", subcore_axis_name="s")
    k = functools.partial(
        pl.kernel,
        mesh=mesh,
        out_type=jax.ShapeDtypeStruct((_HIST, 4, 128, 8, 128), jnp.float32),
        scratch_types=[
            pltpu.VMEM((2, _CHUNK), jnp.int32),
            pltpu.VMEM((2, _CHUNK, _EMBED), jnp.float32),
            pltpu.VMEM((2, _GCHUNK, 4, 8, _LPAD), jnp.float32),
            pltpu.SemaphoreType.DMA,
            pltpu.SemaphoreType.DMA,
            pltpu.SemaphoreType.DMA,
            pltpu.SemaphoreType.DMA,
            pltpu.SemaphoreType.DMA,
            pltpu.SemaphoreType.DMA,
        ],
        compiler_params=pltpu.CompilerParams(
            use_tc_tiling_on_sc=False, needs_layout_passes=False),
    )(_embed_kernel)
    return k(x_flat, table)


def kernel(x, table):
    x_flat = x.T.reshape(-1).astype(jnp.int32)
    out = _embed(x_flat, table)
    return out.transpose(2, 4, 0, 1, 3).reshape(_BATCH, _HIST, _EMBED)


# speedup vs baseline: 3.6243x; 1.1617x over previous
import functools

import jax
import jax.numpy as jnp
from jax import lax
from jax.experimental import pallas as pl
from jax.experimental.pallas import tpu as pltpu
from jax.experimental.pallas import tpu_sc as plsc

_BATCH = 16384
_HIST = 200
_EMBED = 32
_B = _BATCH * _HIST

_NC = 2
_NS = 16
_NW = _NC * _NS

_GROUPS = _B // 128
_G_PER_W = _GROUPS // _NW
_GCHUNK = 4
_CHUNK = _GCHUNK * 128
_N_CHUNKS = _G_PER_W // _GCHUNK
_N_QUADS = _N_CHUNKS // 4
_LPAD = 133


def _embed_kernel(idx_hbm, table_hbm, out_hbm, idx_v, rows_v, embt_v,
                  si0, si1, si2, si3, sg0, sg1, ss0, ss1):
    wid = lax.axis_index("s") * _NC + lax.axis_index("c")
    g_base = wid * _G_PER_W
    si = (si0, si1, si2, si3)
    sg = (sg0, sg1)
    ss = (ss0, ss1)
    iota = lax.broadcasted_iota(jnp.int32, (16,), 0)
    rvec = [(iota + 16 * m) // 8 for m in range(2)]
    svec = [(iota + 16 * m) % 8 for m in range(2)]
    sidx = [rvec[m] * 1064 + svec[m] * 133 for m in range(2)]
    zero = jnp.zeros((16,), jnp.int32)

    def idx_cp(k, t):
        return pltpu.make_async_copy(
            idx_hbm.at[pl.ds((g_base + k * _GCHUNK) * 128, _CHUNK)],
            idx_v.at[t], si[t])

    def gat_cp(b, t):
        return pltpu.make_async_copy(
            table_hbm.at[idx_v.at[t]], rows_v.at[b], sg[b])

    def sct_cp(k, b, g, r):
        g0 = g_base + k * _GCHUNK
        tr = g0 // 1024
        c = (g0 // 8) % 128
        h0 = 8 * tr + g0 % 8
        return pltpu.make_async_copy(
            embt_v.at[b, g, r, :, pl.ds(0, 128)],
            out_hbm.at[h0 + g, r, c], ss[b])

    def transpose(b):
        for g in range(_GCHUNK):
            init = tuple(sidx[m] + dl for dl in range(4) for m in range(2))
            def l_body(q, idxs, g=g):
                vecs = []
                for dl in range(4):
                    l = q * 4 + dl
                    for m in range(2):
                        vecs.append(rows_v[b, g * 128 + l, pl.ds(16 * m, 16)])
                for u in range(8):
                    plsc.store_scatter(embt_v.at[b, g],
                                       [zero, zero, idxs[u]], vecs[u])
                return tuple(v + 4 for v in idxs)
            lax.fori_loop(0, 32, l_body, init)

    def start_sct(k, b):
        for g in range(_GCHUNK):
            for r in range(4):
                sct_cp(k, b, g, r).start()

    def wait_sct(b):
        for g in range(_GCHUNK):
            for r in range(4):
                sct_cp(0, b, g, r).wait()

    for t in range(4):
        idx_cp(t, t).start()
    idx_cp(0, 0).wait()
    gat_cp(0, 0).start()
    idx_cp(1, 1).wait()
    gat_cp(1, 1).start()
    for t in range(4):
        b = t % 2
        gat_cp(b, t).wait()
        idx_cp(t + 4, t).start()
        if t >= 2:
            wait_sct(b)
        transpose(b)
        start_sct(t, b)
        idx_cp(t + 2, (t + 2) % 4).wait()
        gat_cp(b, (t + 2) % 4).start()

    def body(i, carry):
        k0 = 4 * i
        for t in range(4):
            b = t % 2
            gat_cp(b, t).wait()
            idx_cp(k0 + t + 4, t).start()
            wait_sct(b)
            transpose(b)
            start_sct(k0 + t, b)
            idx_cp(k0 + t + 2, (t + 2) % 4).wait()
            gat_cp(b, (t + 2) % 4).start()
        return carry

    lax.fori_loop(1, _N_QUADS - 1, body, 0)

    k0 = 4 * (_N_QUADS - 1)
    for t in range(4):
        b = t % 2
        gat_cp(b, t).wait()
        wait_sct(b)
        transpose(b)
        start_sct(k0 + t, b)
        if t < 2:
            idx_cp(k0 + t + 2, (t + 2) % 4).wait()
            gat_cp(b, (t + 2) % 4).start()
    wait_sct(0)
    wait_sct(1)


@jax.jit
def _embed(x_flat, table):
    mesh = plsc.VectorSubcoreMesh(core_axis_name="c", subcore_axis_name="s")
    k = functools.partial(
        pl.kernel,
        mesh=mesh,
        out_type=jax.ShapeDtypeStruct((_HIST, 4, 128, 8, 128), jnp.float32),
        scratch_types=[
            pltpu.VMEM((4, _CHUNK), jnp.int32),
            pltpu.VMEM((2, _CHUNK, _EMBED), jnp.float32),
            pltpu.VMEM((2, _GCHUNK, 4, 8, _LPAD), jnp.float32),
            pltpu.SemaphoreType.DMA,
            pltpu.SemaphoreType.DMA,
            pltpu.SemaphoreType.DMA,
            pltpu.SemaphoreType.DMA,
            pltpu.SemaphoreType.DMA,
            pltpu.SemaphoreType.DMA,
            pltpu.SemaphoreType.DMA,
            pltpu.SemaphoreType.DMA,
        ],
        compiler_params=pltpu.CompilerParams(
            use_tc_tiling_on_sc=False, needs_layout_passes=False),
    )(_embed_kernel)
    return k(x_flat, table)


def kernel(x, table):
    x_flat = (x.reshape(128, 128, 25, 8).transpose(2, 0, 3, 1)
              .reshape(-1).astype(jnp.int32))
    out = _embed(x_flat, table)
    return out.transpose(2, 4, 0, 1, 3).reshape(_BATCH, _HIST, _EMBED)


# speedup vs baseline: 3.6267x; 1.0007x over previous
import functools

import jax
import jax.numpy as jnp
from jax import lax
from jax.experimental import pallas as pl
from jax.experimental.pallas import tpu as pltpu
from jax.experimental.pallas import tpu_sc as plsc

_BATCH = 16384
_HIST = 200
_EMBED = 32
_B = _BATCH * _HIST

_NC = 2
_NS = 16
_NW = _NC * _NS

_GROUPS = _B // 128
_G_PER_W = _GROUPS // _NW
_GCHUNK = 4
_CHUNK = _GCHUNK * 128
_N_CHUNKS = _G_PER_W // _GCHUNK
_N_QUADS = _N_CHUNKS // 4
_LPAD = 133


def _embed_kernel(idx_hbm, table_hbm, out_hbm, idx_v, rows_v, embt_v,
                  si0, si1, si2, si3, sg0, sg1, ss0, ss1):
    wid = lax.axis_index("s") * _NC + lax.axis_index("c")
    g_base = wid * _G_PER_W
    si = (si0, si1, si2, si3)
    sg = (sg0, sg1)
    ss = (ss0, ss1)
    iota = lax.broadcasted_iota(jnp.int32, (16,), 0)
    rvec = [(iota + 16 * m) // 8 for m in range(2)]
    svec = [(iota + 16 * m) % 8 for m in range(2)]

    def idx_cp(k, t):
        return pltpu.make_async_copy(
            idx_hbm.at[pl.ds((g_base + k * _GCHUNK) * 128, _CHUNK)],
            idx_v.at[t], si[t])

    def gat_cp(b, t=0):
        return pltpu.make_async_copy(
            table_hbm.at[idx_v.at[t]], rows_v.at[b], sg[b])

    def sct_cp(k, b, j):
        g0 = g_base + k * _GCHUNK
        h = g0 // 128
        c0 = g0 % 128
        return pltpu.make_async_copy(
            embt_v.at[b, j, :, :, pl.ds(0, 128)],
            out_hbm.at[h, :, c0 + j], ss[b])

    sidx = [rvec[m] * 1064 + svec[m] * 133 for m in range(2)]
    zero = jnp.zeros((16,), jnp.int32)

    def transpose(b):
        for j in range(_GCHUNK):
            init = tuple(sidx[m] + dl for dl in range(4) for m in range(2))
            def l_body(q, idxs, j=j):
                vecs = []
                for dl in range(4):
                    l = q * 4 + dl
                    for m in range(2):
                        vecs.append(rows_v[b, j * 128 + l, pl.ds(16 * m, 16)])
                for t in range(8):
                    plsc.store_scatter(embt_v.at[b, j],
                                       [zero, zero, idxs[t]], vecs[t])
                return tuple(v + 4 for v in idxs)
            lax.fori_loop(0, 32, l_body, init)

    def start_sct(k, b):
        for j in range(_GCHUNK):
            sct_cp(k, b, j).start()

    def wait_sct(b):
        for j in range(_GCHUNK):
            sct_cp(0, b, j).wait()

    for t in range(4):
        idx_cp(t, t).start()
    idx_cp(0, 0).wait()
    gat_cp(0, 0).start()
    idx_cp(1, 1).wait()
    gat_cp(1, 1).start()
    for t in range(4):
        b = t % 2
        gat_cp(b, t).wait()
        idx_cp(t + 4, t).start()
        if t >= 2:
            wait_sct(b)
        transpose(b)
        start_sct(t, b)
        idx_cp(t + 2, (t + 2) % 4).wait()
        gat_cp(b, (t + 2) % 4).start()

    def body(i, carry):
        k0 = 4 * i
        for t in range(4):
            b = t % 2
            gat_cp(b, t).wait()
            idx_cp(k0 + t + 4, t).start()
            wait_sct(b)
            transpose(b)
            start_sct(k0 + t, b)
            idx_cp(k0 + t + 2, (t + 2) % 4).wait()
            gat_cp(b, (t + 2) % 4).start()
        return carry

    lax.fori_loop(1, _N_QUADS - 1, body, 0)

    k0 = 4 * (_N_QUADS - 1)
    for t in range(4):
        b = t % 2
        gat_cp(b, t).wait()
        wait_sct(b)
        transpose(b)
        start_sct(k0 + t, b)
        if t < 2:
            idx_cp(k0 + t + 2, (t + 2) % 4).wait()
            gat_cp(b, (t + 2) % 4).start()
    wait_sct(0)
    wait_sct(1)


@jax.jit
def _embed(x_flat, table):
    mesh = plsc.VectorSubcoreMesh(core_axis_name="c", subcore_axis_name="s")
    k = functools.partial(
        pl.kernel,
        mesh=mesh,
        out_type=jax.ShapeDtypeStruct((_HIST, 4, 128, 8, 128), jnp.float32),
        scratch_types=[
            pltpu.VMEM((4, _CHUNK), jnp.int32),
            pltpu.VMEM((2, _CHUNK, _EMBED), jnp.float32),
            pltpu.VMEM((2, _GCHUNK, 4, 8, _LPAD), jnp.float32),
            pltpu.SemaphoreType.DMA,
            pltpu.SemaphoreType.DMA,
            pltpu.SemaphoreType.DMA,
            pltpu.SemaphoreType.DMA,
            pltpu.SemaphoreType.DMA,
            pltpu.SemaphoreType.DMA,
            pltpu.SemaphoreType.DMA,
            pltpu.SemaphoreType.DMA,
        ],
        compiler_params=pltpu.CompilerParams(
            use_tc_tiling_on_sc=False, needs_layout_passes=False),
    )(_embed_kernel)
    return k(x_flat, table)


def kernel(x, table):
    x_flat = x.T.reshape(-1).astype(jnp.int32)
    out = _embed(x_flat, table)
    return out.transpose(2, 4, 0, 1, 3).reshape(_BATCH, _HIST, _EMBED)
